# Initial kernel scaffold; baseline (speedup 1.0000x reference)
import functools

import jax
import jax.numpy as jnp
from jax import lax
from jax.experimental import pallas as pl
from jax.experimental.pallas import tpu as pltpu
import jax.experimental.pallas.tpu_sc as plsc

_NC = 2
_NS = 16
_NW = _NC * _NS

_ROWS_PER_BLOCK = 256


def _argmax_body(x_ref, o_ref):
    o_ref[0, 0, :] = jnp.argmax(x_ref[...], axis=1).astype(jnp.int32)


def _make_argmax(bn, v):
    nb = bn // _ROWS_PER_BLOCK
    return pl.pallas_call(
        _argmax_body,
        grid=(nb,),
        in_specs=[pl.BlockSpec((_ROWS_PER_BLOCK, v), lambda i: (i, 0))],
        out_specs=pl.BlockSpec((1, 1, _ROWS_PER_BLOCK), lambda i: (i, 0, 0)),
        out_shape=jax.ShapeDtypeStruct((nb, 1, _ROWS_PER_BLOCK), jnp.int32),
    )


def _table_body(psi_hbm, permt_hbm, t_hbm, idx_v, rows_v, sem):
    wid = lax.axis_index("s") * _NC + lax.axis_index("c")
    nj = permt_hbm.shape[0] // _NW
    pltpu.sync_copy(permt_hbm.at[pl.ds(wid * nj, nj)], idx_v)
    for j in range(nj):
        pltpu.async_copy(psi_hbm.at[idx_v.at[j]], rows_v, sem).wait()
        pltpu.sync_copy(
            rows_v, t_hbm.at[pl.ds((wid * nj + j) * 128, 128)])


def _make_table(v, g, k):
    gv_pad = ((v * g + 128 * _NW - 1) // (128 * _NW)) * (128 * _NW)
    nj = gv_pad // (128 * _NW)
    mesh = plsc.VectorSubcoreMesh(
        core_axis_name="c", subcore_axis_name="s",
        num_cores=_NC, num_subcores=_NS)
    return pl.kernel(
        _table_body,
        out_type=jax.ShapeDtypeStruct((gv_pad, k), jnp.float32),
        mesh=mesh,
        scratch_types=[
            pltpu.VMEM((nj, 128), jnp.int32),
            pltpu.VMEM((128, k), jnp.float32),
            pltpu.SemaphoreType.DMA,
        ],
    ), gv_pad


_CHUNK = 64


def _gather_body(t2_hbm, idx_hbm, out_hbm, idx_v, rows_v, sem):
    wid = lax.axis_index("s") * _NC + lax.axis_index("c")
    bn = out_hbm.shape[0]
    rows_per_tile = bn // _NW
    nc = rows_per_tile // _CHUNK
    pltpu.sync_copy(idx_hbm.at[pl.ds(wid * nc, nc)], idx_v)
    base = wid * rows_per_tile
    for c in range(nc):
        pltpu.async_copy(t2_hbm.at[idx_v.at[c]], rows_v, sem).wait()
        pltpu.sync_copy(rows_v, out_hbm.at[pl.ds(base + c * _CHUNK, _CHUNK)])


def _make_gather(bn, vpad, gk):
    mesh = plsc.VectorSubcoreMesh(
        core_axis_name="c", subcore_axis_name="s",
        num_cores=_NC, num_subcores=_NS)
    nc = bn // _NW // _CHUNK
    return pl.kernel(
        _gather_body,
        out_type=jax.ShapeDtypeStruct((bn, gk), jnp.float32),
        mesh=mesh,
        scratch_types=[
            pltpu.VMEM((nc, _CHUNK), jnp.int32),
            pltpu.VMEM((_CHUNK, gk), jnp.float32),
            pltpu.SemaphoreType.DMA,
        ],
    )


@jax.jit
def kernel(s, perm, psi):
    b, n, v, _ = s.shape
    g = perm.shape[0]
    k = psi.shape[1]
    bn = b * n

    idx = _make_argmax(bn, v)(s.reshape(bn, v)).reshape(bn)

    table_fn, gv_pad = _make_table(v, g, k)
    permt = jnp.pad(perm.T.reshape(v * g), (0, gv_pad - v * g))
    t = table_fn(psi, permt.reshape(gv_pad // 128, 128))
    t2 = t.reshape(gv_pad // g, g * k)

    out = _make_gather(bn, gv_pad // g, g * k)(
        t2, idx.reshape(bn // _CHUNK, _CHUNK))
    return out.reshape(b, n, g, k)


# speedup vs baseline: 1.9204x; 1.9204x over previous
import functools

import jax
import jax.numpy as jnp
from jax import lax
from jax.experimental import pallas as pl
from jax.experimental.pallas import tpu as pltpu
import jax.experimental.pallas.tpu_sc as plsc

_NC = 2
_NS = 16
_NW = _NC * _NS

_ROWS_PER_BLOCK = 256


def _argmax_body(x_ref, o_ref):
    o_ref[0, 0, :] = jnp.argmax(x_ref[...], axis=1).astype(jnp.int32)


def _make_argmax(bn, v):
    nb = bn // _ROWS_PER_BLOCK
    return pl.pallas_call(
        _argmax_body,
        grid=(nb,),
        in_specs=[pl.BlockSpec((_ROWS_PER_BLOCK, v), lambda i: (i, 0))],
        out_specs=pl.BlockSpec((1, 1, _ROWS_PER_BLOCK), lambda i: (i, 0, 0)),
        out_shape=jax.ShapeDtypeStruct((nb, 1, _ROWS_PER_BLOCK), jnp.int32),
    )


def _table_body(psi_hbm, permt_hbm, t_hbm, idx_v, rows_v, sem):
    wid = lax.axis_index("s") * _NC + lax.axis_index("c")
    nj = permt_hbm.shape[0] // _NW
    pltpu.sync_copy(permt_hbm.at[pl.ds(wid * nj, nj)], idx_v)
    for j in range(nj):
        pltpu.async_copy(psi_hbm.at[idx_v.at[j]], rows_v, sem).wait()
        pltpu.sync_copy(
            rows_v, t_hbm.at[pl.ds((wid * nj + j) * 128, 128)])


def _make_table(v, g, k):
    gv_pad = ((v * g + 128 * _NW - 1) // (128 * _NW)) * (128 * _NW)
    nj = gv_pad // (128 * _NW)
    mesh = plsc.VectorSubcoreMesh(
        core_axis_name="c", subcore_axis_name="s",
        num_cores=_NC, num_subcores=_NS)
    return pl.kernel(
        _table_body,
        out_type=jax.ShapeDtypeStruct((gv_pad, k), jnp.float32),
        mesh=mesh,
        compiler_params=pltpu.CompilerParams(use_tc_tiling_on_sc=False),
        scratch_types=[
            pltpu.VMEM((nj, 128), jnp.int32),
            pltpu.VMEM((128, k), jnp.float32),
            pltpu.SemaphoreType.DMA,
        ],
    ), gv_pad


_CHUNK = 64


def _gather_body(t2_hbm, idx_hbm, out_hbm, idx_v, rows_v, sem):
    wid = lax.axis_index("s") * _NC + lax.axis_index("c")
    bn = out_hbm.shape[0]
    rows_per_tile = bn // _NW
    nc = rows_per_tile // _CHUNK
    pltpu.sync_copy(idx_hbm.at[pl.ds(wid * nc, nc)], idx_v)
    base = wid * rows_per_tile
    for c in range(nc):
        pltpu.async_copy(t2_hbm.at[idx_v.at[c]], rows_v, sem).wait()
        pltpu.sync_copy(rows_v, out_hbm.at[pl.ds(base + c * _CHUNK, _CHUNK)])


def _make_gather(bn, vpad, gk):
    mesh = plsc.VectorSubcoreMesh(
        core_axis_name="c", subcore_axis_name="s",
        num_cores=_NC, num_subcores=_NS)
    nc = bn // _NW // _CHUNK
    return pl.kernel(
        _gather_body,
        out_type=jax.ShapeDtypeStruct((bn, gk), jnp.float32),
        mesh=mesh,
        compiler_params=pltpu.CompilerParams(use_tc_tiling_on_sc=False),
        scratch_types=[
            pltpu.VMEM((nc, _CHUNK), jnp.int32),
            pltpu.VMEM((_CHUNK, gk), jnp.float32),
            pltpu.SemaphoreType.DMA,
        ],
    )


@jax.jit
def kernel(s, perm, psi):
    b, n, v, _ = s.shape
    g = perm.shape[0]
    k = psi.shape[1]
    bn = b * n

    idx = _make_argmax(bn, v)(s.reshape(bn, v)).reshape(bn)

    table_fn, gv_pad = _make_table(v, g, k)
    permt = jnp.pad(perm.T.reshape(v * g), (0, gv_pad - v * g))
    t = table_fn(psi, permt.reshape(gv_pad // 128, 128))
    t2 = t.reshape(gv_pad // g, g * k)

    out = _make_gather(bn, gv_pad // g, g * k)(
        t2, idx.reshape(bn // _CHUNK, _CHUNK))
    return out.reshape(b, n, g, k)


# speedup vs baseline: 6.0568x; 3.1540x over previous
import jax
import jax.numpy as jnp
from jax import lax
from jax.experimental import pallas as pl
from jax.experimental.pallas import tpu as pltpu
import jax.experimental.pallas.tpu_sc as plsc

_NC = 2
_NS = 16
_NW = _NC * _NS

_B, _N, _V, _G, _K = 1024, 20, 1000, 8, 64
_VP = 1024


def _argmax_body(x_ref, o_ref):
    base = lax.broadcasted_iota(jnp.int32, (8, _B), 0)
    m = x_ref[0, pl.ds(0, 8), 0, :]
    vidx = base
    for t in range(1, _V // 8):
        x = x_ref[0, pl.ds(t * 8, 8), 0, :]
        better = x > m
        m = jnp.where(better, x, m)
        vidx = jnp.where(better, base + t * 8, vidx)
    mx = jnp.max(m, axis=0)
    idx = jnp.min(jnp.where(m == mx[None], vidx, _V), axis=0)
    o_ref[0, 0] = idx


def _make_argmax():
    return pl.pallas_call(
        _argmax_body,
        grid=(_N,),
        in_specs=[pl.BlockSpec((1, _V, 1, _B), lambda i: (i, 0, 0, 0))],
        out_specs=pl.BlockSpec((1, 1, _B), lambda i: (i, 0, 0)),
        out_shape=jax.ShapeDtypeStruct((_N, 1, _B), jnp.int32),
    )


def _table_body(psi_hbm, permt_hbm, t_hbm, idx_v, rows_v, skew_v, sem):
    wid = lax.axis_index("s") * _NC + lax.axis_index("c")
    nj = permt_hbm.shape[0] // _NW
    pltpu.sync_copy(permt_hbm.at[pl.ds(wid * nj, nj)], idx_v)
    lanes = lax.broadcasted_iota(jnp.int32, (16,), 0)
    for j in range(nj):
        pltpu.async_copy(psi_hbm.at[idx_v.at[j]], rows_v, sem).wait()
        base = wid * nj * 128 + j * 128

        def skew_row(i, carry):
            row = jnp.full((16,), i, jnp.int32)
            rot = jnp.bitwise_and(base + i, _K - 1)
            for l in range(_K // 16):
                src = jnp.bitwise_xor(lanes + l * 16, rot)
                skew_v[i, pl.ds(l * 16, 16)] = plsc.load_gather(
                    rows_v, [row, src])
            return carry

        lax.fori_loop(0, 128, skew_row, 0)
        pltpu.sync_copy(skew_v, t_hbm.at[pl.ds(base, 128)])


def _make_table():
    gv = _G * _VP
    nj = gv // (128 * _NW)
    mesh = plsc.VectorSubcoreMesh(
        core_axis_name="c", subcore_axis_name="s",
        num_cores=_NC, num_subcores=_NS)
    return pl.kernel(
        _table_body,
        out_type=jax.ShapeDtypeStruct((gv, _K), jnp.float32),
        mesh=mesh,
        compiler_params=pltpu.CompilerParams(
            use_tc_tiling_on_sc=False, needs_layout_passes=False),
        scratch_types=[
            pltpu.VMEM((nj, 128), jnp.int32),
            pltpu.VMEM((128, _K), jnp.float32),
            pltpu.VMEM((128, _K), jnp.float32),
            pltpu.SemaphoreType.DMA,
        ],
    )


def _gather_body(tp_hbm, idx_hbm, out_hbm, idx_v, tg_v, buf_v, sem):
    wid = lax.axis_index("s") * _NC + lax.axis_index("c")
    g = lax.rem(wid, _G)
    chunk = lax.div(wid, _G)
    bt0 = chunk * 2
    pltpu.sync_copy(tp_hbm.at[pl.ds(g * (_VP // 2), _VP // 2)], tg_v)
    pltpu.sync_copy(idx_hbm.at[:, pl.ds(bt0, 2), :], idx_v)

    def n_body(n, carry):
        slot = lax.rem(n, 2)
        for bh in range(2):
            for bv in range(8):
                idxv = idx_v[n, bh, pl.ds(bv * 16, 16)]
                rowv = lax.shift_right_logical(idxv, 1)
                colx = lax.shift_left(jnp.bitwise_and(idxv, 1), 6) | (
                    jnp.bitwise_and(idxv, _K - 1))
                for k0 in range(0, _K, 8):
                    vals = [plsc.load_gather(
                        tg_v, [rowv, jnp.bitwise_xor(colx, k0 + u)])
                            for u in range(8)]
                    for u in range(8):
                        buf_v[slot, bh, k0 + u, pl.ds(bv * 16, 16)] = vals[u]
        @pl.when(n >= 2)
        def _():
            pltpu.make_async_copy(
                buf_v.at[slot, 0],
                out_hbm.at[n - 2, g, :, pl.ds(bt0 * 128, 128)], sem).wait()
            pltpu.make_async_copy(
                buf_v.at[slot, 1],
                out_hbm.at[n - 2, g, :, pl.ds(bt0 * 128 + 128, 128)],
                sem).wait()
        for bh in range(2):
            pltpu.async_copy(
                buf_v.at[slot, bh],
                out_hbm.at[n, g, :, pl.ds((bt0 + bh) * 128, 128)], sem)
        return carry

    lax.fori_loop(0, _N, n_body, 0)
    for m in range(_N - 2, _N):
        slot = m % 2
        for bh in range(2):
            pltpu.make_async_copy(
                buf_v.at[slot, bh],
                out_hbm.at[m, g, :, pl.ds((bt0 + bh) * 128, 128)],
                sem).wait()


def _make_gather():
    mesh = plsc.VectorSubcoreMesh(
        core_axis_name="c", subcore_axis_name="s",
        num_cores=_NC, num_subcores=_NS)
    return pl.kernel(
        _gather_body,
        out_type=jax.ShapeDtypeStruct((_N, _G, _K, _B), jnp.float32),
        mesh=mesh,
        compiler_params=pltpu.CompilerParams(needs_layout_passes=False),
        scratch_types=[
            pltpu.VMEM((_N, 2, 128), jnp.int32),
            pltpu.VMEM((_VP // 2, 2 * _K), jnp.float32),
            pltpu.VMEM((2, 2, _K, 128), jnp.float32),
            pltpu.SemaphoreType.DMA,
        ],
    )


@jax.jit
def kernel(s, perm, psi):
    b, n, v, _ = s.shape
    g = perm.shape[0]
    k = psi.shape[1]

    s_t = jnp.transpose(s[..., 0:1], (1, 2, 3, 0))
    idx = _make_argmax()(s_t).reshape(n, 8, 128)

    permp = jnp.pad(perm, ((0, 0), (0, _VP - v))).reshape(
        g * _VP // 128, 128)
    t = _make_table()(psi, permp)
    tp = t.reshape(g * _VP // 2, 2 * k)

    outp = _make_gather()(tp, idx)
    return jnp.transpose(outp, (3, 0, 1, 2))
